# Initial kernel scaffold; baseline (speedup 1.0000x reference)
import jax
import jax.numpy as jnp
from jax.experimental import pallas as pl


def kernel(des_a, des_g, centroids_a, centroids_g, W1, b1, W2, b2):
    raise NotImplementedError("write your pallas kernel here")


# speedup vs baseline: 1.6923x; 1.6923x over previous
import jax
import jax.numpy as jnp
from jax import lax
from jax.experimental import pallas as pl
from jax.experimental.pallas import tpu as pltpu

_B, _N, _D, _K, _C = 64, 512, 128, 512, 11


def _score_hist(des, cents):
    cnorm = jnp.sum(cents * cents, axis=-1)
    dnorm = jnp.sum(des * des, axis=-1, keepdims=True)
    dot = lax.dot_general(des, cents, (((1,), (1,)), ((), ())))
    d2 = dnorm - 2.0 * dot + cnorm
    idx = jnp.argmin(d2, axis=-1)
    kk = lax.broadcasted_iota(jnp.int32, (_N, _K), 1)
    onehot = (idx[:, None] == kk).astype(jnp.float32)
    return jnp.sum(onehot, axis=0) * (1.0 / _N)


def _body(da_ref, dg_ref, ca_ref, cg_ref, w1_ref, b1_ref, w2_ref, b2_ref,
          out_ref, hist_ref):
    b = pl.program_id(0)
    ha = _score_hist(da_ref[0], ca_ref[...])
    hg = _score_hist(dg_ref[0], cg_ref[...])
    hist_ref[pl.ds(b, 1), :] = jnp.concatenate([ha, hg])[None, :]

    @pl.when(b == _B - 1)
    def _():
        hist = hist_ref[...]
        h = lax.dot_general(hist, w1_ref[...], (((1,), (1,)), ((), ())))
        h = jnp.maximum(h + b1_ref[...][None, :], 0.0)
        logits = lax.dot_general(h, w2_ref[...], (((1,), (1,)), ((), ())))
        out_ref[...] = logits + b2_ref[...][None, :]


def kernel(des_a, des_g, centroids_a, centroids_g, W1, b1, W2, b2):
    return pl.pallas_call(
        _body,
        grid=(_B,),
        in_specs=[
            pl.BlockSpec((1, _N, _D), lambda b: (b, 0, 0)),
            pl.BlockSpec((1, _N, _D), lambda b: (b, 0, 0)),
            pl.BlockSpec((_K, _D), lambda b: (0, 0)),
            pl.BlockSpec((_K, _D), lambda b: (0, 0)),
            pl.BlockSpec((_K, 2 * _K), lambda b: (0, 0)),
            pl.BlockSpec((_K,), lambda b: (0,)),
            pl.BlockSpec((_C, _K), lambda b: (0, 0)),
            pl.BlockSpec((_C,), lambda b: (0,)),
        ],
        out_specs=pl.BlockSpec((_B, _C), lambda b: (0, 0)),
        out_shape=jax.ShapeDtypeStruct((_B, _C), jnp.float32),
        scratch_shapes=[pltpu.VMEM((_B, 2 * _K), jnp.float32)],
        compiler_params=pltpu.CompilerParams(
            dimension_semantics=("arbitrary",),
        ),
    )(des_a, des_g, centroids_a, centroids_g, W1, b1, W2, b2)


# speedup vs baseline: 3.3077x; 1.9546x over previous
import jax
import jax.numpy as jnp
from jax import lax
from jax.experimental import pallas as pl
from jax.experimental.pallas import tpu as pltpu

_B, _N, _D, _K, _C = 64, 512, 128, 512, 11
_BB = 8


def _score_hist(des, cents):
    cnorm = jnp.sum(cents * cents, axis=-1)
    dnorm = jnp.sum(des * des, axis=-1, keepdims=True)
    dot = lax.dot_general(des, cents, (((1,), (1,)), ((), ())))
    d2 = dnorm - 2.0 * dot + cnorm
    kk = lax.broadcasted_iota(jnp.int32, (_BB * _N, _K), 1)
    m = jnp.min(d2, axis=-1, keepdims=True)
    idx = jnp.min(jnp.where(d2 == m, kk, _K), axis=-1, keepdims=True)
    onehot = (kk == idx).astype(jnp.float32)
    return jnp.sum(onehot.reshape(_BB, _N, _K), axis=1) * (1.0 / _N)


def _body(da_ref, dg_ref, ca_ref, cg_ref, w1_ref, b1_ref, w2_ref, b2_ref,
          out_ref, hist_ref):
    b = pl.program_id(0)
    ha = _score_hist(da_ref[...].reshape(_BB * _N, _D), ca_ref[...])
    hg = _score_hist(dg_ref[...].reshape(_BB * _N, _D), cg_ref[...])
    hist_ref[pl.ds(pl.multiple_of(b * _BB, _BB), _BB), :] = jnp.concatenate([ha, hg], axis=-1)

    @pl.when(b == _B // _BB - 1)
    def _():
        hist = hist_ref[...]
        h = lax.dot_general(hist, w1_ref[...], (((1,), (1,)), ((), ())))
        h = jnp.maximum(h + b1_ref[...][None, :], 0.0)
        logits = lax.dot_general(h, w2_ref[...], (((1,), (1,)), ((), ())))
        out_ref[...] = logits + b2_ref[...][None, :]


def kernel(des_a, des_g, centroids_a, centroids_g, W1, b1, W2, b2):
    return pl.pallas_call(
        _body,
        grid=(_B // _BB,),
        in_specs=[
            pl.BlockSpec((_BB, _N, _D), lambda b: (b, 0, 0)),
            pl.BlockSpec((_BB, _N, _D), lambda b: (b, 0, 0)),
            pl.BlockSpec((_K, _D), lambda b: (0, 0)),
            pl.BlockSpec((_K, _D), lambda b: (0, 0)),
            pl.BlockSpec((_K, 2 * _K), lambda b: (0, 0)),
            pl.BlockSpec((_K,), lambda b: (0,)),
            pl.BlockSpec((_C, _K), lambda b: (0, 0)),
            pl.BlockSpec((_C,), lambda b: (0,)),
        ],
        out_specs=pl.BlockSpec((_B, _C), lambda b: (0, 0)),
        out_shape=jax.ShapeDtypeStruct((_B, _C), jnp.float32),
        scratch_shapes=[pltpu.VMEM((_B, 2 * _K), jnp.float32)],
        compiler_params=pltpu.CompilerParams(
            dimension_semantics=("arbitrary",),
        ),
    )(des_a, des_g, centroids_a, centroids_g, W1, b1, W2, b2)


# speedup vs baseline: 5.6596x; 1.7110x over previous
import jax
import jax.numpy as jnp
from jax import lax
from jax.experimental import pallas as pl
from jax.experimental.pallas import tpu as pltpu

_B, _N, _D, _K, _C = 64, 512, 128, 512, 11
_BB = 8
_R = _BB * _N


def _hist_rows(des, cneg, cnorm, out_sl):
    dnorm = jnp.sum(des * des, axis=-1, keepdims=True)
    dot = lax.dot_general(des, cneg, (((1,), (1,)), ((), ())))
    d2 = (dnorm + dot) + cnorm
    m = jnp.min(d2, axis=-1, keepdims=True)
    mask = (d2 == m).astype(jnp.float32)
    cnt = jnp.sum(mask.reshape(_BB, _N, _K), axis=1)
    out_sl[...] = cnt * (1.0 / _N)
    total = jnp.sum(cnt)

    @pl.when(total != float(_R))
    def _():
        kk = lax.broadcasted_iota(jnp.int32, (_R, _K), 1)
        idx = jnp.min(jnp.where(d2 == m, kk, _K), axis=-1, keepdims=True)
        onehot = (kk == idx).astype(jnp.float32)
        out_sl[...] = jnp.sum(onehot.reshape(_BB, _N, _K), axis=1) * (1.0 / _N)


def _body(da_ref, dg_ref, ca_ref, cg_ref, w1_ref, b1_ref, w2_ref, b2_ref,
          out_ref, hist_ref):
    b = pl.program_id(0)
    row0 = pl.multiple_of(b * _BB, _BB)
    ca = ca_ref[...]
    cg = cg_ref[...]
    _hist_rows(da_ref[...].reshape(_R, _D), -2.0 * ca,
               jnp.sum(ca * ca, axis=-1),
               hist_ref.at[pl.ds(row0, _BB), pl.ds(0, _K)])
    _hist_rows(dg_ref[...].reshape(_R, _D), -2.0 * cg,
               jnp.sum(cg * cg, axis=-1),
               hist_ref.at[pl.ds(row0, _BB), pl.ds(_K, _K)])

    @pl.when(b == _B // _BB - 1)
    def _():
        hist = hist_ref[...]
        h = lax.dot_general(hist, w1_ref[...], (((1,), (1,)), ((), ())))
        h = jnp.maximum(h + b1_ref[...][None, :], 0.0)
        logits = lax.dot_general(h, w2_ref[...], (((1,), (1,)), ((), ())))
        out_ref[...] = logits + b2_ref[...][None, :]


def kernel(des_a, des_g, centroids_a, centroids_g, W1, b1, W2, b2):
    return pl.pallas_call(
        _body,
        grid=(_B // _BB,),
        in_specs=[
            pl.BlockSpec((_BB, _N, _D), lambda b: (b, 0, 0)),
            pl.BlockSpec((_BB, _N, _D), lambda b: (b, 0, 0)),
            pl.BlockSpec((_K, _D), lambda b: (0, 0)),
            pl.BlockSpec((_K, _D), lambda b: (0, 0)),
            pl.BlockSpec((_K, 2 * _K), lambda b: (0, 0)),
            pl.BlockSpec((_K,), lambda b: (0,)),
            pl.BlockSpec((_C, _K), lambda b: (0, 0)),
            pl.BlockSpec((_C,), lambda b: (0,)),
        ],
        out_specs=pl.BlockSpec((_B, _C), lambda b: (0, 0)),
        out_shape=jax.ShapeDtypeStruct((_B, _C), jnp.float32),
        scratch_shapes=[pltpu.VMEM((_B, 2 * _K), jnp.float32)],
        compiler_params=pltpu.CompilerParams(
            dimension_semantics=("arbitrary",),
        ),
    )(des_a, des_g, centroids_a, centroids_g, W1, b1, W2, b2)
